# Initial kernel scaffold; baseline (speedup 1.0000x reference)
import jax
import jax.numpy as jnp
from jax.experimental import pallas as pl


def kernel(x, noise):
    raise NotImplementedError("write your pallas kernel here")


# speedup vs baseline: 1.3155x; 1.3155x over previous
import jax
import jax.numpy as jnp
from jax import lax
from jax.experimental import pallas as pl
from jax.experimental.pallas import tpu as pltpu

_P = 8
_MASK_RATIO = 0.75
_CC = 16


def _fused_kernel(noise_j_ref, noise_k_ref, x_ref, out_ref, mask_ref, spat_ref):
    nc = pl.program_id(1)
    np_ = noise_j_ref.shape[1]
    hp = x_ref.shape[2] // _P
    num_keep = int(np_ * (1.0 - _MASK_RATIO))

    @pl.when(nc == 0)
    def _compute_mask():
        nj = noise_j_ref[0]
        nk = noise_k_ref[0]
        j_idx = lax.broadcasted_iota(jnp.int32, (np_, np_), 0)
        k_idx = lax.broadcasted_iota(jnp.int32, (np_, np_), 1)
        lt = nk < nj
        tie = (nk == nj) & (k_idx < j_idx)
        rank = jnp.sum((lt | tie).astype(jnp.float32), axis=1, keepdims=True)
        m = (rank < num_keep).astype(jnp.float32)
        mask_ref[0] = m

        h_full = spat_ref.shape[0]
        a_i = lax.broadcasted_iota(jnp.int32, (h_full, np_), 0)
        a_p = lax.broadcasted_iota(jnp.int32, (h_full, np_), 1)
        a_sel = ((a_p // hp) == (a_i // _P)).astype(jnp.float32)
        b_p = lax.broadcasted_iota(jnp.int32, (np_, h_full), 0)
        b_j = lax.broadcasted_iota(jnp.int32, (np_, h_full), 1)
        b_sel = ((b_p % hp) == (b_j // _P)).astype(jnp.float32)
        spat_ref[...] = jnp.dot(a_sel, m * b_sel,
                                preferred_element_type=jnp.float32)

    out_ref[...] = x_ref[...] * spat_ref[...][None, None, :, :]


def kernel(x, noise):
    b, c, h_full, w_full = x.shape
    num_patches = noise.shape[1]
    nc = c // _CC

    noise_j = noise.reshape(b, num_patches, 1)
    noise_k = noise.reshape(b, 1, num_patches)

    x_img, mask3 = pl.pallas_call(
        _fused_kernel,
        grid=(b, nc),
        in_specs=[
            pl.BlockSpec((1, num_patches, 1), lambda i, j: (i, 0, 0)),
            pl.BlockSpec((1, 1, num_patches), lambda i, j: (i, 0, 0)),
            pl.BlockSpec((1, _CC, h_full, w_full), lambda i, j: (i, j, 0, 0)),
        ],
        out_specs=[
            pl.BlockSpec((1, _CC, h_full, w_full), lambda i, j: (i, j, 0, 0)),
            pl.BlockSpec((1, num_patches, 1), lambda i, j: (i, 0, 0)),
        ],
        out_shape=[
            jax.ShapeDtypeStruct((b, c, h_full, w_full), x.dtype),
            jax.ShapeDtypeStruct((b, num_patches, 1), jnp.float32),
        ],
        scratch_shapes=[pltpu.VMEM((h_full, w_full), jnp.float32)],
        compiler_params=pltpu.CompilerParams(
            dimension_semantics=("arbitrary", "arbitrary"),
        ),
    )(noise_j, noise_k, x)

    return (x_img, mask3.reshape(b, num_patches))
